# baseline (speedup 1.0000x reference)
import jax
import jax.numpy as jnp
import numpy as np
from jax.experimental import pallas as pl

_B, _N, _K, _D = 8, 2048, 16, 128
_TILE = 512


def _knn_body(lrow, lall, lxc, lyc, lxr, lyr, ew, bt, idx_out, attr_out):
    b = pl.program_id(0)
    i = pl.program_id(1)

    xc = lxc[0]
    yc = lyc[0]
    xr = lxr[0]
    yr = lyr[0]

    dot = jax.lax.dot_general(
        lrow[0], lall[0],
        dimension_numbers=(((1,), (1,)), ((), ())),
        preferred_element_type=jnp.float32,
        precision=jax.lax.Precision.DEFAULT,
    )
    sqc = xc * xc + yc * yc
    sqr = xr * xr + yr * yr
    d2 = (sqc + sqr) - 2.0 * dot

    row_g = jax.lax.broadcasted_iota(jnp.int32, (_TILE, _N), 0) + i * _TILE
    col = jax.lax.broadcasted_iota(jnp.int32, (_TILE, _N), 1)
    vals = jnp.where(row_g == col, jnp.float32(1e38), jnp.maximum(d2, 1e-12))

    key = jax.lax.bitcast_convert_type(
        (jax.lax.bitcast_convert_type(vals, jnp.int32)
         & jnp.int32(~0x7FF)) | col, jnp.float32)

    koi = jax.lax.broadcasted_iota(jnp.int32, (_TILE, _K), 1)
    ksel = jnp.zeros((_TILE, _K), jnp.float32)
    imax = jnp.float32(3e38)

    def bitonic_merge(a, b):
        x = a + b[::-1]
        s = len(x) // 2
        while s >= 1:
            for base in range(0, len(x), 2 * s):
                for i in range(base, base + s):
                    x[i], x[i + s] = (jnp.minimum(x[i], x[i + s]),
                                      jnp.maximum(x[i], x[i + s]))
            s //= 2
        return x

    runs = [[key[:, c * 128:(c + 1) * 128]] for c in range(16)]
    while len(runs) > 1:
        runs = [bitonic_merge(runs[2 * i], runs[2 * i + 1])
                for i in range(len(runs) // 2)]
    c = runs[0]

    for t in range(_K):
        m = jnp.min(c[0], axis=1, keepdims=True)
        eq = c[0] == m
        depth = _K - t
        for k in range(depth - 1):
            c[k] = jnp.where(eq, c[k + 1], c[k])
        c[depth - 1] = jnp.where(eq, imax, c[depth - 1])
        ksel = jnp.where(koi == t, m, ksel)

    kseli = jax.lax.bitcast_convert_type(ksel, jnp.int32)
    isel = kseli & jnp.int32(0x7FF)
    dsel = jax.lax.bitcast_convert_type(kseli & jnp.int32(~0x7FF), jnp.float32)

    idx_out[0] = isel + b * _N
    ed = jnp.sqrt(dsel)
    attr = jax.lax.dot_general(
        ed, ew[...],
        dimension_numbers=(((1,), (0,)), ((), ())),
        preferred_element_type=jnp.float32,
        precision=jax.lax.Precision.DEFAULT,
    )
    attr_out[0] = attr + bt[...]


def kernel(locs, init_embeddings, W, b):
    Bv, Nv, _ = locs.shape
    lxc = locs[:, :, 0:1]
    lyc = locs[:, :, 1:2]
    lxr = locs[:, :, 0].reshape(Bv, 1, Nv)
    lyr = locs[:, :, 1].reshape(Bv, 1, Nv)
    Wv = W.reshape(_D)
    ew = jnp.kron(jnp.eye(_K, dtype=jnp.float32), Wv[None, :])
    bt = jnp.tile(b, _K)[None, :]

    grid = (Bv, Nv // _TILE)
    idx_out, attr_out = pl.pallas_call(
        _knn_body,
        grid=grid,
        in_specs=[
            pl.BlockSpec((1, _TILE, 2), lambda b_, i: (b_, i, 0)),
            pl.BlockSpec((1, Nv, 2), lambda b_, i: (b_, 0, 0)),
            pl.BlockSpec((1, _TILE, 1), lambda b_, i: (b_, i, 0)),
            pl.BlockSpec((1, _TILE, 1), lambda b_, i: (b_, i, 0)),
            pl.BlockSpec((1, 1, Nv), lambda b_, i: (b_, 0, 0)),
            pl.BlockSpec((1, 1, Nv), lambda b_, i: (b_, 0, 0)),
            pl.BlockSpec((_K, _K * _D), lambda b_, i: (0, 0)),
            pl.BlockSpec((1, _K * _D), lambda b_, i: (0, 0)),
        ],
        out_specs=[
            pl.BlockSpec((1, _TILE, _K), lambda b_, i: (b_, i, 0)),
            pl.BlockSpec((1, _TILE, _K * _D), lambda b_, i: (b_, i, 0)),
        ],
        out_shape=[
            jax.ShapeDtypeStruct((Bv, Nv, _K), jnp.int32),
            jax.ShapeDtypeStruct((Bv, Nv, _K * _D), jnp.float32),
        ],
    )(locs, locs, lxc, lyc, lxr, lyr, ew, bt)

    x = init_embeddings.reshape(Bv * Nv, _D)
    src = jnp.broadcast_to(
        jnp.arange(Bv * Nv, dtype=jnp.int32)[:, None], (Bv * Nv, _K)
    ).reshape(-1)
    dst = idx_out.reshape(-1)
    edge_index = jnp.stack([src, dst], axis=0)
    edge_attr = attr_out.reshape(Bv * Nv * _K, _D)
    return x, edge_index, edge_attr


# speedup vs baseline: 1.0282x; 1.0282x over previous
import jax
import jax.numpy as jnp
import numpy as np
from jax.experimental import pallas as pl

_B, _N, _K, _D = 8, 2048, 16, 128
_TILE = 512


def _knn_body(lrow, lall, lxc, lyc, lxr, lyr, ew, bt, idx_out, attr_out):
    b = pl.program_id(0)
    i = pl.program_id(1)

    xc = lxc[0]
    yc = lyc[0]
    xr = lxr[0]
    yr = lyr[0]

    dot = jax.lax.dot_general(
        lrow[0], lall[0],
        dimension_numbers=(((1,), (1,)), ((), ())),
        preferred_element_type=jnp.float32,
        precision=jax.lax.Precision.DEFAULT,
    )
    sqc = xc * xc + yc * yc
    sqr = xr * xr + yr * yr
    d2 = (sqc + sqr) - 2.0 * dot

    row_g = jax.lax.broadcasted_iota(jnp.int32, (_TILE, _N), 0) + i * _TILE
    col = jax.lax.broadcasted_iota(jnp.int32, (_TILE, _N), 1)
    vals = jnp.where(row_g == col, jnp.float32(1e38), jnp.maximum(d2, 1e-12))

    key = jax.lax.bitcast_convert_type(
        (jax.lax.bitcast_convert_type(vals, jnp.int32)
         & jnp.int32(~0x7FF)) | col, jnp.float32)

    koi = jax.lax.broadcasted_iota(jnp.int32, (_TILE, _K), 1)
    ksel = jnp.zeros((_TILE, _K), jnp.float32)
    imax = jnp.float32(3e38)

    def bitonic_merge(a, b):
        x = a + b[::-1]
        s = len(x) // 2
        while s >= 1:
            for base in range(0, len(x), 2 * s):
                for i in range(base, base + s):
                    x[i], x[i + s] = (jnp.minimum(x[i], x[i + s]),
                                      jnp.maximum(x[i], x[i + s]))
            s //= 2
        return x

    def merge_low(a, b):
        x = [jnp.minimum(a[i], b[len(b) - 1 - i]) for i in range(len(a))]
        s = len(x) // 2
        while s >= 1:
            for base in range(0, len(x), 2 * s):
                for i in range(base, base + s):
                    x[i], x[i + s] = (jnp.minimum(x[i], x[i + s]),
                                      jnp.maximum(x[i], x[i + s]))
            s //= 2
        return x

    cols = [key[:, c * 128:(c + 1) * 128] for c in range(16)]
    m_last = None
    for batch in range(_K // 8):
        cf = (cols if m_last is None
              else [jnp.where(cc > m_last, cc, imax) for cc in cols])
        runs = [[cc] for cc in cf]
        while len(runs) > 2:
            runs = [bitonic_merge(runs[2 * i], runs[2 * i + 1])
                    for i in range(len(runs) // 2)]
        c = merge_low(runs[0], runs[1])
        for t in range(8):
            m = jnp.min(c[0], axis=1, keepdims=True)
            eq = c[0] == m
            depth = 8 - t
            for k in range(depth - 1):
                c[k] = jnp.where(eq, c[k + 1], c[k])
            c[depth - 1] = jnp.where(eq, imax, c[depth - 1])
            ksel = jnp.where(koi == (8 * batch + t), m, ksel)
            m_last = m

    kseli = jax.lax.bitcast_convert_type(ksel, jnp.int32)
    isel = kseli & jnp.int32(0x7FF)
    dsel = jax.lax.bitcast_convert_type(kseli & jnp.int32(~0x7FF), jnp.float32)

    idx_out[0] = isel + b * _N
    ed = jnp.sqrt(dsel)
    attr = jax.lax.dot_general(
        ed, ew[...],
        dimension_numbers=(((1,), (0,)), ((), ())),
        preferred_element_type=jnp.float32,
        precision=jax.lax.Precision.DEFAULT,
    )
    attr_out[0] = attr + bt[...]


def kernel(locs, init_embeddings, W, b):
    Bv, Nv, _ = locs.shape
    lxc = locs[:, :, 0:1]
    lyc = locs[:, :, 1:2]
    lxr = locs[:, :, 0].reshape(Bv, 1, Nv)
    lyr = locs[:, :, 1].reshape(Bv, 1, Nv)
    Wv = W.reshape(_D)
    ew = jnp.kron(jnp.eye(_K, dtype=jnp.float32), Wv[None, :])
    bt = jnp.tile(b, _K)[None, :]

    grid = (Bv, Nv // _TILE)
    idx_out, attr_out = pl.pallas_call(
        _knn_body,
        grid=grid,
        in_specs=[
            pl.BlockSpec((1, _TILE, 2), lambda b_, i: (b_, i, 0)),
            pl.BlockSpec((1, Nv, 2), lambda b_, i: (b_, 0, 0)),
            pl.BlockSpec((1, _TILE, 1), lambda b_, i: (b_, i, 0)),
            pl.BlockSpec((1, _TILE, 1), lambda b_, i: (b_, i, 0)),
            pl.BlockSpec((1, 1, Nv), lambda b_, i: (b_, 0, 0)),
            pl.BlockSpec((1, 1, Nv), lambda b_, i: (b_, 0, 0)),
            pl.BlockSpec((_K, _K * _D), lambda b_, i: (0, 0)),
            pl.BlockSpec((1, _K * _D), lambda b_, i: (0, 0)),
        ],
        out_specs=[
            pl.BlockSpec((1, _TILE, _K), lambda b_, i: (b_, i, 0)),
            pl.BlockSpec((1, _TILE, _K * _D), lambda b_, i: (b_, i, 0)),
        ],
        out_shape=[
            jax.ShapeDtypeStruct((Bv, Nv, _K), jnp.int32),
            jax.ShapeDtypeStruct((Bv, Nv, _K * _D), jnp.float32),
        ],
    )(locs, locs, lxc, lyc, lxr, lyr, ew, bt)

    x = init_embeddings.reshape(Bv * Nv, _D)
    src = jnp.broadcast_to(
        jnp.arange(Bv * Nv, dtype=jnp.int32)[:, None], (Bv * Nv, _K)
    ).reshape(-1)
    dst = idx_out.reshape(-1)
    edge_index = jnp.stack([src, dst], axis=0)
    edge_attr = attr_out.reshape(Bv * Nv * _K, _D)
    return x, edge_index, edge_attr
